# baseline (speedup 1.0000x reference)
import functools

import jax
import jax.numpy as jnp
from jax import lax
from jax.experimental import pallas as pl
from jax.experimental.pallas import tpu as pltpu
from jax.experimental.pallas import tpu_sc as plsc

N = 10000
E = 320000
D = 128
H = 128
DE = 16
PW = 16

NC = 2
NS = 16
NW = NC * NS
EPW = E // NW
CH = 128
NFULL = EPW // CH
TAIL = EPW - NFULL * CH
NP = 10240
RPT = NP // NS

EB = 6400
NB = 2000


def _sc_gather(table, row, col, width, tiled, ecount):
    epw = ecount // NW
    nfull = epw // CH
    tail = epw - nfull * CH
    pairs = nfull // 2
    rem = nfull - 2 * pairs
    mesh = plsc.VectorSubcoreMesh(core_axis_name="c", subcore_axis_name="s")

    @functools.partial(
        pl.kernel,
        mesh=mesh,
        compiler_params=pltpu.CompilerParams(use_tc_tiling_on_sc=tiled),
        out_type=(jax.ShapeDtypeStruct((ecount, width), jnp.float32),
                  jax.ShapeDtypeStruct((ecount, width), jnp.float32)),
        scratch_types=[pltpu.VMEM((4, CH), jnp.int32),
                       pltpu.VMEM((2, CH, width), jnp.float32),
                       pltpu.VMEM((2, CH, width), jnp.float32),
                       pltpu.VMEM((2, tail), jnp.int32),
                       pltpu.VMEM((2, tail, width), jnp.float32),
                       pltpu.SemaphoreType.DMA,
                       pltpu.SemaphoreType.DMA,
                       pltpu.SemaphoreType.DMA,
                       pltpu.SemaphoreType.DMA,
                       pltpu.SemaphoreType.DMA,
                       pltpu.SemaphoreType.DMA],
    )
    def k(table_hbm, row_hbm, col_hbm, xi_hbm, xj_hbm,
          idxs, bufr, bufc, tidx, tbuf, si0, si1, sg0, sg1, sw0, sw1):
        wid = lax.axis_index("s") * NC + lax.axis_index("c")
        wbase = wid * epw
        si = (si0, si1)
        sg = (sg0, sg1)
        sw = (sw0, sw1)

        def issue_idx(c, p):
            base = pl.multiple_of(wbase + c * CH, 8)
            pltpu.async_copy(row_hbm.at[pl.ds(base, CH)], idxs.at[2 * p], si[p])
            pltpu.async_copy(col_hbm.at[pl.ds(base, CH)], idxs.at[2 * p + 1],
                             si[p])

        def wait_idx(p):
            pltpu.make_async_copy(row_hbm.at[pl.ds(0, CH)], idxs.at[2 * p],
                                  si[p]).wait()
            pltpu.make_async_copy(row_hbm.at[pl.ds(0, CH)], idxs.at[2 * p + 1],
                                  si[p]).wait()

        def issue_gather(p):
            pltpu.async_copy(table_hbm.at[idxs.at[2 * p]], bufr.at[p], sg[p])
            pltpu.async_copy(table_hbm.at[idxs.at[2 * p + 1]], bufc.at[p],
                             sg[p])

        def wait_gather(p):
            pltpu.make_async_copy(table_hbm.at[idxs.at[2 * p]], bufr.at[p],
                                  sg[p]).wait()
            pltpu.make_async_copy(table_hbm.at[idxs.at[2 * p + 1]],
                                  bufc.at[p], sg[p]).wait()

        def issue_wb(c, p):
            base = pl.multiple_of(wbase + c * CH, 8)
            pltpu.async_copy(bufr.at[p], xi_hbm.at[pl.ds(base, CH)], sw[p])
            pltpu.async_copy(bufc.at[p], xj_hbm.at[pl.ds(base, CH)], sw[p])

        def wait_wb(p):
            pltpu.make_async_copy(bufr.at[p], xi_hbm.at[pl.ds(0, CH)],
                                  sw[p]).wait()
            pltpu.make_async_copy(bufc.at[p], xj_hbm.at[pl.ds(0, CH)],
                                  sw[p]).wait()

        issue_idx(0, 0)
        issue_idx(1, 1)

        def body(kk, carry):
            for p in (0, 1):
                c = 2 * kk + p

                @pl.when(c >= 2)
                def _():
                    wait_wb(p)

                wait_idx(p)
                issue_gather(p)
                wait_gather(p)

                @pl.when(c + 2 < nfull)
                def _():
                    issue_idx(c + 2, p)

                issue_wb(c, p)
            return carry

        lax.fori_loop(0, pairs, body, 0)
        if rem:
            c = nfull - 1
            wait_wb(0)
            wait_idx(0)
            issue_gather(0)
            wait_gather(0)
            issue_wb(c, 0)
        wait_wb(0)
        wait_wb(1)

        tb = pl.multiple_of(wbase + nfull * CH, 8)
        pltpu.sync_copy(row_hbm.at[pl.ds(tb, tail)], tidx.at[0])
        pltpu.sync_copy(col_hbm.at[pl.ds(tb, tail)], tidx.at[1])
        cp1 = pltpu.async_copy(table_hbm.at[tidx.at[0]], tbuf.at[0], sg0)
        cp2 = pltpu.async_copy(table_hbm.at[tidx.at[1]], tbuf.at[1], sg1)
        cp1.wait()
        cp2.wait()
        pltpu.sync_copy(tbuf.at[0], xi_hbm.at[pl.ds(tb, tail)])
        pltpu.sync_copy(tbuf.at[1], xj_hbm.at[pl.ds(tb, tail)])

    return k(table, row, col)


def _sc_scatter(vals, row, zeros_tile, width, tiled, ecount):
    epw = ecount // NW
    nfull = epw // CH
    tail = epw - nfull * CH
    pairs = nfull // 2
    rem = nfull - 2 * pairs
    mesh = plsc.VectorSubcoreMesh(core_axis_name="c", subcore_axis_name="s")

    @functools.partial(
        pl.kernel,
        mesh=mesh,
        compiler_params=pltpu.CompilerParams(use_tc_tiling_on_sc=tiled),
        out_type=jax.ShapeDtypeStruct((NC * NP, width), jnp.float32),
        scratch_types=[pltpu.VMEM((2, CH), jnp.int32),
                       pltpu.VMEM((2, CH, width), jnp.float32),
                       pltpu.VMEM((tail,), jnp.int32),
                       pltpu.VMEM((tail, width), jnp.float32),
                       pltpu.VMEM_SHARED((NP, width), jnp.float32),
                       pltpu.SemaphoreType.DMA,
                       pltpu.SemaphoreType.DMA],
    )
    def k(m_hbm, row_hbm, z_hbm, out_hbm, idxs, bufs, tidx, tbuf, acc_sh,
          sr0, sr1):
        c_ax = lax.axis_index("c")
        s_ax = lax.axis_index("s")
        pltpu.sync_copy(z_hbm, acc_sh.at[pl.ds(s_ax * RPT, RPT)])
        plsc.subcore_barrier()

        wbase = (c_ax * NS + s_ax) * epw
        sr = (sr0, sr1)

        def issue_rd(c, p):
            base = pl.multiple_of(wbase + c * CH, 8)
            pltpu.async_copy(row_hbm.at[pl.ds(base, CH)], idxs.at[p], sr[p])
            pltpu.async_copy(m_hbm.at[pl.ds(base, CH)], bufs.at[p], sr[p])

        def wait_rd(p):
            pltpu.make_async_copy(row_hbm.at[pl.ds(0, CH)], idxs.at[p],
                                  sr[p]).wait()
            pltpu.make_async_copy(m_hbm.at[pl.ds(0, CH)], bufs.at[p],
                                  sr[p]).wait()

        issue_rd(0, 0)
        issue_rd(1, 1)

        def body(kk, carry):
            for p in (0, 1):
                c = 2 * kk + p
                wait_rd(p)
                pltpu.sync_copy(bufs.at[p], acc_sh.at[idxs.at[p]], add=True)

                @pl.when(c + 2 < nfull)
                def _():
                    issue_rd(c + 2, p)
            return carry

        lax.fori_loop(0, pairs, body, 0)
        if rem:
            wait_rd(0)
            pltpu.sync_copy(bufs.at[0], acc_sh.at[idxs.at[0]], add=True)

        tb = pl.multiple_of(wbase + nfull * CH, 8)
        pltpu.sync_copy(row_hbm.at[pl.ds(tb, tail)], tidx)
        pltpu.sync_copy(m_hbm.at[pl.ds(tb, tail)], tbuf)
        pltpu.sync_copy(tbuf, acc_sh.at[tidx], add=True)

        plsc.subcore_barrier()
        obase = pl.multiple_of(c_ax * NP + s_ax * RPT, 8)
        pltpu.sync_copy(acc_sh.at[pl.ds(s_ax * RPT, RPT)],
                        out_hbm.at[pl.ds(obase, RPT)])

    return k(vals, row, zeros_tile)


def _tc_edge(xi, xj, pi, pj, edge_attr,
             w1a, w1b, w1r, w1e, b1, w2, b2, wc1, bc1, wc2r, bc2):
    ecount = xi.shape[0]
    grid = ecount // EB

    def body(xi_ref, xj_ref, pi_ref, pj_ref, ea_ref,
             w1a_ref, w1b_ref, w1r_ref, w1e_ref, b1_ref, w2_ref, b2_ref,
             wc1_ref, bc1_ref, wc2_ref, bc2_ref, m_ref, t_ref):
        S = 8
        SB = EB // S
        for s in range(S):
            sl = pl.ds(s * SB, SB)
            pdiff = pi_ref[sl, :] - pj_ref[sl, :]
            radial = jnp.sum(pdiff * pdiff, axis=1, keepdims=True)

            m = (jnp.dot(xi_ref[sl, :], w1a_ref[...],
                         preferred_element_type=jnp.float32)
                 + jnp.dot(xj_ref[sl, :], w1b_ref[...],
                           preferred_element_type=jnp.float32)
                 + jnp.dot(ea_ref[sl, :], w1e_ref[...],
                           preferred_element_type=jnp.float32)
                 + radial * w1r_ref[...]
                 + b1_ref[...])
            m = m * jax.nn.sigmoid(m)
            m = (jnp.dot(m, w2_ref[...], preferred_element_type=jnp.float32)
                 + b2_ref[...])
            m = m * jax.nn.sigmoid(m)
            m_ref[sl, :] = m

            cc = (jnp.dot(m, wc1_ref[...], preferred_element_type=jnp.float32)
                  + bc1_ref[...])
            cc = cc * jax.nn.sigmoid(cc)
            cc = jnp.sum(cc * wc2_ref[...], axis=1, keepdims=True) + bc2_ref[...]

            lane = lax.broadcasted_iota(jnp.int32, (SB, PW), 1)
            t_ref[sl, :] = jnp.where(lane == 3,
                                     jnp.float32(1.0), pdiff * cc)

    full = lambda shape: pl.BlockSpec(shape, lambda i: (0,) * len(shape))
    return pl.pallas_call(
        body,
        grid=(grid,),
        in_specs=[
            pl.BlockSpec((EB, D), lambda i: (i, 0)),
            pl.BlockSpec((EB, D), lambda i: (i, 0)),
            pl.BlockSpec((EB, PW), lambda i: (i, 0)),
            pl.BlockSpec((EB, PW), lambda i: (i, 0)),
            pl.BlockSpec((EB, DE), lambda i: (i, 0)),
            full((D, H)), full((D, H)), full((1, H)), full((DE, H)),
            full((1, H)), full((H, H)), full((1, H)),
            full((H, H)), full((1, H)), full((1, H)), full((1, 1)),
        ],
        out_specs=[pl.BlockSpec((EB, D), lambda i: (i, 0)),
                   pl.BlockSpec((EB, PW), lambda i: (i, 0))],
        out_shape=[jax.ShapeDtypeStruct((ecount, D), jnp.float32),
                   jax.ShapeDtypeStruct((ecount, PW), jnp.float32)],
    )(xi, xj, pi, pj, edge_attr,
      w1a, w1b, w1r, w1e, b1, w2, b2, wc1, bc1, wc2r, bc2)


def _tc_node(mp0, mp1, tp0, tp1, x, pos, wna, wnb, bn1, wn2, bn2):
    grid = N // NB

    def body(mp0_ref, mp1_ref, tp0_ref, tp1_ref, x_ref, pos_ref,
             wna_ref, wnb_ref, bn1_ref,
             wn2_ref, bn2_ref, h_ref, np_ref):
        agg = (mp0_ref[0] + mp0_ref[1]) + (mp1_ref[0] + mp1_ref[1])
        tc16 = (tp0_ref[0] + tp0_ref[1]) + (tp1_ref[0] + tp1_ref[1])
        sum_trans = tc16[:, :3]
        counts = tc16[:, 3:4]
        xv = x_ref[...]

        h = (jnp.dot(xv, wna_ref[...], preferred_element_type=jnp.float32)
             + jnp.dot(agg, wnb_ref[...], preferred_element_type=jnp.float32)
             + bn1_ref[...])
        h = h * jax.nn.sigmoid(h)
        h = jnp.dot(h, wn2_ref[...], preferred_element_type=jnp.float32) + bn2_ref[...]
        h_ref[...] = xv + h
        np_ref[...] = pos_ref[...] + sum_trans / jnp.maximum(counts, 1.0)

    full = lambda shape: pl.BlockSpec(shape, lambda i: (0,) * len(shape))
    return pl.pallas_call(
        body,
        grid=(grid,),
        in_specs=[
            pl.BlockSpec((NC, NB, D), lambda i: (0, i, 0)),
            pl.BlockSpec((NC, NB, D), lambda i: (0, i, 0)),
            pl.BlockSpec((NC, NB, PW), lambda i: (0, i, 0)),
            pl.BlockSpec((NC, NB, PW), lambda i: (0, i, 0)),
            pl.BlockSpec((NB, D), lambda i: (i, 0)),
            pl.BlockSpec((NB, 3), lambda i: (i, 0)),
            full((D, H)), full((D, H)), full((1, H)),
            full((H, D)), full((1, D)),
        ],
        out_specs=[pl.BlockSpec((NB, D), lambda i: (i, 0)),
                   pl.BlockSpec((NB, 3), lambda i: (i, 0))],
        out_shape=[jax.ShapeDtypeStruct((N, D), jnp.float32),
                   jax.ShapeDtypeStruct((N, 3), jnp.float32)],
    )(mp0, mp1, tp0, tp1, x, pos, wna, wnb, bn1, wn2, bn2)


def kernel(x, pos, edge_index, edge_attr, W_e1, b_e1, W_e2, b_e2,
           W_c1, b_c1, W_c2, b_c2, W_n1, b_n1, W_n2, b_n2):
    row = edge_index[0]
    col = edge_index[1]

    pos16 = jnp.concatenate([pos, jnp.zeros((N, PW - 3), jnp.float32)], axis=1)

    ew = (W_e1[:D], W_e1[D:2 * D], W_e1[2 * D:2 * D + 1], W_e1[2 * D + 1:],
          b_e1.reshape(1, H), W_e2, b_e2.reshape(1, H),
          W_c1, b_c1.reshape(1, H), W_c2.reshape(1, H), b_c2.reshape(1, 1))
    zm = jnp.zeros((RPT, D), jnp.float32)
    zt = jnp.zeros((RPT, PW), jnp.float32)

    EH = E // 2
    mparts, tparts = [], []
    for h in range(2):
        rh = lax.slice(row, (h * EH,), ((h + 1) * EH,))
        ch = lax.slice(col, (h * EH,), ((h + 1) * EH,))
        eah = lax.slice(edge_attr, (h * EH, 0), ((h + 1) * EH, DE))
        xi, xj = _sc_gather(x, rh, ch, D, True, EH)
        pi, pj = _sc_gather(pos16, rh, ch, PW, False, EH)
        m, tc16 = _tc_edge(xi, xj, pi, pj, eah, *ew)
        mparts.append(_sc_scatter(m, rh, zm, D, True, EH).reshape(NC, NP, D))
        tparts.append(_sc_scatter(tc16, rh, zt, PW, False, EH).reshape(
            NC, NP, PW))

    h_out, new_pos = _tc_node(
        mparts[0], mparts[1], tparts[0], tparts[1], x, pos,
        W_n1[:D], W_n1[D:], b_n1.reshape(1, H), W_n2, b_n2.reshape(1, D))
    return (h_out, new_pos)


# speedup vs baseline: 1.0161x; 1.0161x over previous
import functools

import jax
import jax.numpy as jnp
from jax import lax
from jax.experimental import pallas as pl
from jax.experimental.pallas import tpu as pltpu
from jax.experimental.pallas import tpu_sc as plsc

N = 10000
E = 320000
D = 128
H = 128
DE = 16
PW = 16

NC = 2
NS = 16
NW = NC * NS
EPW = E // NW
CH = 128
NFULL = EPW // CH
TAIL = EPW - NFULL * CH
NP = 10240
RPT = NP // NS

EB = 6400
NB = 2000


def _sc_gather(table, row, col, width, tiled, ecount):
    epw = ecount // NW
    nfull = epw // CH
    tail = epw - nfull * CH
    pairs = nfull // 2
    rem = nfull - 2 * pairs
    mesh = plsc.VectorSubcoreMesh(core_axis_name="c", subcore_axis_name="s")

    @functools.partial(
        pl.kernel,
        mesh=mesh,
        compiler_params=pltpu.CompilerParams(use_tc_tiling_on_sc=tiled),
        out_type=(jax.ShapeDtypeStruct((ecount, width), jnp.float32),
                  jax.ShapeDtypeStruct((ecount, width), jnp.float32)),
        scratch_types=[pltpu.VMEM((4, CH), jnp.int32),
                       pltpu.VMEM((2, CH, width), jnp.float32),
                       pltpu.VMEM((2, CH, width), jnp.float32),
                       pltpu.VMEM((2, tail), jnp.int32),
                       pltpu.VMEM((2, tail, width), jnp.float32),
                       pltpu.SemaphoreType.DMA,
                       pltpu.SemaphoreType.DMA,
                       pltpu.SemaphoreType.DMA,
                       pltpu.SemaphoreType.DMA,
                       pltpu.SemaphoreType.DMA,
                       pltpu.SemaphoreType.DMA],
    )
    def k(table_hbm, row_hbm, col_hbm, xi_hbm, xj_hbm,
          idxs, bufr, bufc, tidx, tbuf, si0, si1, sg0, sg1, sw0, sw1):
        wid = lax.axis_index("s") * NC + lax.axis_index("c")
        wbase = wid * epw
        si = (si0, si1)
        sg = (sg0, sg1)
        sw = (sw0, sw1)

        def issue_idx(c, p):
            base = pl.multiple_of(wbase + c * CH, 8)
            pltpu.async_copy(row_hbm.at[pl.ds(base, CH)], idxs.at[2 * p], si[p])
            pltpu.async_copy(col_hbm.at[pl.ds(base, CH)], idxs.at[2 * p + 1],
                             si[p])

        def wait_idx(p):
            pltpu.make_async_copy(row_hbm.at[pl.ds(0, CH)], idxs.at[2 * p],
                                  si[p]).wait()
            pltpu.make_async_copy(row_hbm.at[pl.ds(0, CH)], idxs.at[2 * p + 1],
                                  si[p]).wait()

        def issue_gather(p):
            pltpu.async_copy(table_hbm.at[idxs.at[2 * p]], bufr.at[p], sg[p])
            pltpu.async_copy(table_hbm.at[idxs.at[2 * p + 1]], bufc.at[p],
                             sg[p])

        def wait_gather(p):
            pltpu.make_async_copy(table_hbm.at[idxs.at[2 * p]], bufr.at[p],
                                  sg[p]).wait()
            pltpu.make_async_copy(table_hbm.at[idxs.at[2 * p + 1]],
                                  bufc.at[p], sg[p]).wait()

        def issue_wb(c, p):
            base = pl.multiple_of(wbase + c * CH, 8)
            pltpu.async_copy(bufr.at[p], xi_hbm.at[pl.ds(base, CH)], sw[p])
            pltpu.async_copy(bufc.at[p], xj_hbm.at[pl.ds(base, CH)], sw[p])

        def wait_wb(p):
            pltpu.make_async_copy(bufr.at[p], xi_hbm.at[pl.ds(0, CH)],
                                  sw[p]).wait()
            pltpu.make_async_copy(bufc.at[p], xj_hbm.at[pl.ds(0, CH)],
                                  sw[p]).wait()

        issue_idx(0, 0)
        issue_idx(1, 1)

        def body(kk, carry):
            for p in (0, 1):
                c = 2 * kk + p

                @pl.when(c >= 2)
                def _():
                    wait_wb(p)

                wait_idx(p)
                issue_gather(p)
                wait_gather(p)

                @pl.when(c + 2 < nfull)
                def _():
                    issue_idx(c + 2, p)

                issue_wb(c, p)
            return carry

        lax.fori_loop(0, pairs, body, 0)
        if rem:
            c = nfull - 1
            wait_wb(0)
            wait_idx(0)
            issue_gather(0)
            wait_gather(0)
            issue_wb(c, 0)
        wait_wb(0)
        wait_wb(1)

        tb = pl.multiple_of(wbase + nfull * CH, 8)
        pltpu.sync_copy(row_hbm.at[pl.ds(tb, tail)], tidx.at[0])
        pltpu.sync_copy(col_hbm.at[pl.ds(tb, tail)], tidx.at[1])
        cp1 = pltpu.async_copy(table_hbm.at[tidx.at[0]], tbuf.at[0], sg0)
        cp2 = pltpu.async_copy(table_hbm.at[tidx.at[1]], tbuf.at[1], sg1)
        cp1.wait()
        cp2.wait()
        pltpu.sync_copy(tbuf.at[0], xi_hbm.at[pl.ds(tb, tail)])
        pltpu.sync_copy(tbuf.at[1], xj_hbm.at[pl.ds(tb, tail)])

    return k(table, row, col)


def _sc_scatter(vals, row, zeros_tile, width, tiled, ecount):
    epw = ecount // NW
    nfull = epw // CH
    tail = epw - nfull * CH
    pairs = nfull // 2
    rem = nfull - 2 * pairs
    mesh = plsc.VectorSubcoreMesh(core_axis_name="c", subcore_axis_name="s")

    @functools.partial(
        pl.kernel,
        mesh=mesh,
        compiler_params=pltpu.CompilerParams(use_tc_tiling_on_sc=tiled),
        out_type=jax.ShapeDtypeStruct((NC * NP, width), jnp.float32),
        scratch_types=[pltpu.VMEM((2, CH), jnp.int32),
                       pltpu.VMEM((2, CH, width), jnp.float32),
                       pltpu.VMEM((tail,), jnp.int32),
                       pltpu.VMEM((tail, width), jnp.float32),
                       pltpu.VMEM_SHARED((NP, width), jnp.float32),
                       pltpu.SemaphoreType.DMA,
                       pltpu.SemaphoreType.DMA],
    )
    def k(m_hbm, row_hbm, z_hbm, out_hbm, idxs, bufs, tidx, tbuf, acc_sh,
          sr0, sr1):
        c_ax = lax.axis_index("c")
        s_ax = lax.axis_index("s")
        pltpu.sync_copy(z_hbm, acc_sh.at[pl.ds(s_ax * RPT, RPT)])
        plsc.subcore_barrier()

        wbase = (c_ax * NS + s_ax) * epw
        sr = (sr0, sr1)

        def issue_rd(c, p):
            base = pl.multiple_of(wbase + c * CH, 8)
            pltpu.async_copy(row_hbm.at[pl.ds(base, CH)], idxs.at[p], sr[p])
            pltpu.async_copy(m_hbm.at[pl.ds(base, CH)], bufs.at[p], sr[p])

        def wait_rd(p):
            pltpu.make_async_copy(row_hbm.at[pl.ds(0, CH)], idxs.at[p],
                                  sr[p]).wait()
            pltpu.make_async_copy(m_hbm.at[pl.ds(0, CH)], bufs.at[p],
                                  sr[p]).wait()

        issue_rd(0, 0)
        issue_rd(1, 1)

        def body(kk, carry):
            for p in (0, 1):
                c = 2 * kk + p
                wait_rd(p)
                pltpu.sync_copy(bufs.at[p], acc_sh.at[idxs.at[p]], add=True)

                @pl.when(c + 2 < nfull)
                def _():
                    issue_rd(c + 2, p)
            return carry

        lax.fori_loop(0, pairs, body, 0)
        if rem:
            wait_rd(0)
            pltpu.sync_copy(bufs.at[0], acc_sh.at[idxs.at[0]], add=True)

        tb = pl.multiple_of(wbase + nfull * CH, 8)
        pltpu.sync_copy(row_hbm.at[pl.ds(tb, tail)], tidx)
        pltpu.sync_copy(m_hbm.at[pl.ds(tb, tail)], tbuf)
        pltpu.sync_copy(tbuf, acc_sh.at[tidx], add=True)

        plsc.subcore_barrier()
        obase = pl.multiple_of(c_ax * NP + s_ax * RPT, 8)
        pltpu.sync_copy(acc_sh.at[pl.ds(s_ax * RPT, RPT)],
                        out_hbm.at[pl.ds(obase, RPT)])

    return k(vals, row, zeros_tile)


def _tc_edge(xi, xj, pi, pj, edge_attr, eoff,
             w1a, w1b, w1r, w1e, b1, w2, b2, wc1, bc1, wc2r, bc2):
    ecount = xi.shape[0]
    grid = ecount // EB
    boff = eoff // EB

    def body(xi_ref, xj_ref, pi_ref, pj_ref, ea_ref,
             w1a_ref, w1b_ref, w1r_ref, w1e_ref, b1_ref, w2_ref, b2_ref,
             wc1_ref, bc1_ref, wc2_ref, bc2_ref, m_ref, t_ref):
        S = 8
        SB = EB // S
        for s in range(S):
            sl = pl.ds(s * SB, SB)
            pdiff = pi_ref[sl, :] - pj_ref[sl, :]
            radial = jnp.sum(pdiff * pdiff, axis=1, keepdims=True)

            m = (jnp.dot(xi_ref[sl, :], w1a_ref[...],
                         preferred_element_type=jnp.float32)
                 + jnp.dot(xj_ref[sl, :], w1b_ref[...],
                           preferred_element_type=jnp.float32)
                 + jnp.dot(ea_ref[sl, :], w1e_ref[...],
                           preferred_element_type=jnp.float32)
                 + radial * w1r_ref[...]
                 + b1_ref[...])
            m = m * jax.nn.sigmoid(m)
            m = (jnp.dot(m, w2_ref[...], preferred_element_type=jnp.float32)
                 + b2_ref[...])
            m = m * jax.nn.sigmoid(m)
            m_ref[sl, :] = m

            cc = (jnp.dot(m, wc1_ref[...], preferred_element_type=jnp.float32)
                  + bc1_ref[...])
            cc = cc * jax.nn.sigmoid(cc)
            cc = jnp.sum(cc * wc2_ref[...], axis=1, keepdims=True) + bc2_ref[...]

            lane = lax.broadcasted_iota(jnp.int32, (SB, PW), 1)
            t_ref[sl, :] = jnp.where(lane == 3,
                                     jnp.float32(1.0), pdiff * cc)

    full = lambda shape: pl.BlockSpec(shape, lambda i: (0,) * len(shape))
    return pl.pallas_call(
        body,
        grid=(grid,),
        in_specs=[
            pl.BlockSpec((EB, D), lambda i: (i, 0)),
            pl.BlockSpec((EB, D), lambda i: (i, 0)),
            pl.BlockSpec((EB, PW), lambda i: (i, 0)),
            pl.BlockSpec((EB, PW), lambda i: (i, 0)),
            pl.BlockSpec((EB, DE), lambda i: (i + boff, 0)),
            full((D, H)), full((D, H)), full((1, H)), full((DE, H)),
            full((1, H)), full((H, H)), full((1, H)),
            full((H, H)), full((1, H)), full((1, H)), full((1, 1)),
        ],
        out_specs=[pl.BlockSpec((EB, D), lambda i: (i, 0)),
                   pl.BlockSpec((EB, PW), lambda i: (i, 0))],
        out_shape=[jax.ShapeDtypeStruct((ecount, D), jnp.float32),
                   jax.ShapeDtypeStruct((ecount, PW), jnp.float32)],
    )(xi, xj, pi, pj, edge_attr,
      w1a, w1b, w1r, w1e, b1, w2, b2, wc1, bc1, wc2r, bc2)


def _tc_node(mp0, mp1, tp0, tp1, x, pos, wna, wnb, bn1, wn2, bn2):
    grid = N // NB

    def body(mp0_ref, mp1_ref, tp0_ref, tp1_ref, x_ref, pos_ref,
             wna_ref, wnb_ref, bn1_ref,
             wn2_ref, bn2_ref, h_ref, np_ref):
        agg = (mp0_ref[0] + mp0_ref[1]) + (mp1_ref[0] + mp1_ref[1])
        tc16 = (tp0_ref[0] + tp0_ref[1]) + (tp1_ref[0] + tp1_ref[1])
        sum_trans = tc16[:, :3]
        counts = tc16[:, 3:4]
        xv = x_ref[...]

        h = (jnp.dot(xv, wna_ref[...], preferred_element_type=jnp.float32)
             + jnp.dot(agg, wnb_ref[...], preferred_element_type=jnp.float32)
             + bn1_ref[...])
        h = h * jax.nn.sigmoid(h)
        h = jnp.dot(h, wn2_ref[...], preferred_element_type=jnp.float32) + bn2_ref[...]
        h_ref[...] = xv + h
        np_ref[...] = pos_ref[...] + sum_trans / jnp.maximum(counts, 1.0)

    full = lambda shape: pl.BlockSpec(shape, lambda i: (0,) * len(shape))
    return pl.pallas_call(
        body,
        grid=(grid,),
        in_specs=[
            pl.BlockSpec((NC, NB, D), lambda i: (0, i, 0)),
            pl.BlockSpec((NC, NB, D), lambda i: (0, i, 0)),
            pl.BlockSpec((NC, NB, PW), lambda i: (0, i, 0)),
            pl.BlockSpec((NC, NB, PW), lambda i: (0, i, 0)),
            pl.BlockSpec((NB, D), lambda i: (i, 0)),
            pl.BlockSpec((NB, 3), lambda i: (i, 0)),
            full((D, H)), full((D, H)), full((1, H)),
            full((H, D)), full((1, D)),
        ],
        out_specs=[pl.BlockSpec((NB, D), lambda i: (i, 0)),
                   pl.BlockSpec((NB, 3), lambda i: (i, 0))],
        out_shape=[jax.ShapeDtypeStruct((N, D), jnp.float32),
                   jax.ShapeDtypeStruct((N, 3), jnp.float32)],
    )(mp0, mp1, tp0, tp1, x, pos, wna, wnb, bn1, wn2, bn2)


def kernel(x, pos, edge_index, edge_attr, W_e1, b_e1, W_e2, b_e2,
           W_c1, b_c1, W_c2, b_c2, W_n1, b_n1, W_n2, b_n2):
    row = edge_index[0]
    col = edge_index[1]

    pos16 = jnp.concatenate([pos, jnp.zeros((N, PW - 3), jnp.float32)], axis=1)

    ew = (W_e1[:D], W_e1[D:2 * D], W_e1[2 * D:2 * D + 1], W_e1[2 * D + 1:],
          b_e1.reshape(1, H), W_e2, b_e2.reshape(1, H),
          W_c1, b_c1.reshape(1, H), W_c2.reshape(1, H), b_c2.reshape(1, 1))
    zm = jnp.zeros((RPT, D), jnp.float32)
    zt = jnp.zeros((RPT, PW), jnp.float32)

    EH = E // 2
    rh, ch, gx, gp = [], [], [], []
    for h in range(2):
        rh.append(lax.slice(row, (h * EH,), ((h + 1) * EH,)))
        ch.append(lax.slice(col, (h * EH,), ((h + 1) * EH,)))
        gx.append(_sc_gather(x, rh[h], ch[h], D, True, EH))
        gp.append(_sc_gather(pos16, rh[h], ch[h], PW, False, EH))
    mparts, tparts = [], []
    for h in range(2):
        (xi, xj), (pi, pj) = gx[h], gp[h]
        m, tc16 = _tc_edge(xi, xj, pi, pj, edge_attr, h * EH, *ew)
        mparts.append(_sc_scatter(m, rh[h], zm, D, True, EH).reshape(
            NC, NP, D))
        tparts.append(_sc_scatter(tc16, rh[h], zt, PW, False, EH).reshape(
            NC, NP, PW))

    h_out, new_pos = _tc_node(
        mparts[0], mparts[1], tparts[0], tparts[1], x, pos,
        W_n1[:D], W_n1[D:], b_n1.reshape(1, H), W_n2, b_n2.reshape(1, D))
    return (h_out, new_pos)


# speedup vs baseline: 1.0614x; 1.0446x over previous
import functools

import jax
import jax.numpy as jnp
from jax import lax
from jax.experimental import pallas as pl
from jax.experimental.pallas import tpu as pltpu
from jax.experimental.pallas import tpu_sc as plsc

N = 10000
E = 320000
D = 128
H = 128
DE = 16
PW = 16

NC = 2
NS = 16
NW = NC * NS
EPW = E // NW
CH = 128
NFULL = EPW // CH
TAIL = EPW - NFULL * CH
NP = 10240
RPT = NP // NS

EB = 6400
NB = 2000


def _sc_gather(table, row, col, width, tiled, ecount):
    epw = ecount // NW
    nfull = epw // CH
    tail = epw - nfull * CH
    pairs = nfull // 2
    rem = nfull - 2 * pairs
    mesh = plsc.VectorSubcoreMesh(core_axis_name="c", subcore_axis_name="s")

    @functools.partial(
        pl.kernel,
        mesh=mesh,
        compiler_params=pltpu.CompilerParams(use_tc_tiling_on_sc=tiled),
        out_type=(jax.ShapeDtypeStruct((ecount, width), jnp.float32),
                  jax.ShapeDtypeStruct((ecount, width), jnp.float32)),
        scratch_types=[pltpu.VMEM((4, CH), jnp.int32),
                       pltpu.VMEM((2, CH, width), jnp.float32),
                       pltpu.VMEM((2, CH, width), jnp.float32),
                       pltpu.VMEM((2, tail), jnp.int32),
                       pltpu.VMEM((2, tail, width), jnp.float32),
                       pltpu.SemaphoreType.DMA,
                       pltpu.SemaphoreType.DMA,
                       pltpu.SemaphoreType.DMA,
                       pltpu.SemaphoreType.DMA,
                       pltpu.SemaphoreType.DMA,
                       pltpu.SemaphoreType.DMA],
    )
    def k(table_hbm, row_hbm, col_hbm, xi_hbm, xj_hbm,
          idxs, bufr, bufc, tidx, tbuf, si0, si1, sg0, sg1, sw0, sw1):
        wid = lax.axis_index("s") * NC + lax.axis_index("c")
        wbase = wid * epw
        si = (si0, si1)
        sg = (sg0, sg1)
        sw = (sw0, sw1)

        def issue_idx(c, p):
            base = pl.multiple_of(wbase + c * CH, 8)
            pltpu.async_copy(row_hbm.at[pl.ds(base, CH)], idxs.at[2 * p], si[p])
            pltpu.async_copy(col_hbm.at[pl.ds(base, CH)], idxs.at[2 * p + 1],
                             si[p])

        def wait_idx(p):
            pltpu.make_async_copy(row_hbm.at[pl.ds(0, CH)], idxs.at[2 * p],
                                  si[p]).wait()
            pltpu.make_async_copy(row_hbm.at[pl.ds(0, CH)], idxs.at[2 * p + 1],
                                  si[p]).wait()

        def issue_gather(p):
            pltpu.async_copy(table_hbm.at[idxs.at[2 * p]], bufr.at[p], sg[p])
            pltpu.async_copy(table_hbm.at[idxs.at[2 * p + 1]], bufc.at[p],
                             sg[p])

        def wait_gather(p):
            pltpu.make_async_copy(table_hbm.at[idxs.at[2 * p]], bufr.at[p],
                                  sg[p]).wait()
            pltpu.make_async_copy(table_hbm.at[idxs.at[2 * p + 1]],
                                  bufc.at[p], sg[p]).wait()

        def issue_wb(c, p):
            base = pl.multiple_of(wbase + c * CH, 8)
            pltpu.async_copy(bufr.at[p], xi_hbm.at[pl.ds(base, CH)], sw[p])
            pltpu.async_copy(bufc.at[p], xj_hbm.at[pl.ds(base, CH)], sw[p])

        def wait_wb(p):
            pltpu.make_async_copy(bufr.at[p], xi_hbm.at[pl.ds(0, CH)],
                                  sw[p]).wait()
            pltpu.make_async_copy(bufc.at[p], xj_hbm.at[pl.ds(0, CH)],
                                  sw[p]).wait()

        issue_idx(0, 0)
        issue_idx(1, 1)

        def body(kk, carry):
            for p in (0, 1):
                c = 2 * kk + p

                @pl.when(c >= 2)
                def _():
                    wait_wb(p)

                wait_idx(p)
                issue_gather(p)
                wait_gather(p)

                @pl.when(c + 2 < nfull)
                def _():
                    issue_idx(c + 2, p)

                issue_wb(c, p)
            return carry

        lax.fori_loop(0, pairs, body, 0)
        if rem:
            c = nfull - 1
            wait_wb(0)
            wait_idx(0)
            issue_gather(0)
            wait_gather(0)
            issue_wb(c, 0)
        wait_wb(0)
        wait_wb(1)

        tb = pl.multiple_of(wbase + nfull * CH, 8)
        pltpu.sync_copy(row_hbm.at[pl.ds(tb, tail)], tidx.at[0])
        pltpu.sync_copy(col_hbm.at[pl.ds(tb, tail)], tidx.at[1])
        cp1 = pltpu.async_copy(table_hbm.at[tidx.at[0]], tbuf.at[0], sg0)
        cp2 = pltpu.async_copy(table_hbm.at[tidx.at[1]], tbuf.at[1], sg1)
        cp1.wait()
        cp2.wait()
        pltpu.sync_copy(tbuf.at[0], xi_hbm.at[pl.ds(tb, tail)])
        pltpu.sync_copy(tbuf.at[1], xj_hbm.at[pl.ds(tb, tail)])

    return k(table, row, col)


def _sc_scatter(vals, row, zeros_tile, width, tiled, ecount):
    epw = ecount // NW
    nfull = epw // CH
    tail = epw - nfull * CH
    pairs = nfull // 2
    rem = nfull - 2 * pairs
    mesh = plsc.VectorSubcoreMesh(core_axis_name="c", subcore_axis_name="s")

    @functools.partial(
        pl.kernel,
        mesh=mesh,
        compiler_params=pltpu.CompilerParams(use_tc_tiling_on_sc=tiled),
        out_type=jax.ShapeDtypeStruct((NC * NP, width), jnp.float32),
        scratch_types=[pltpu.VMEM((2, CH), jnp.int32),
                       pltpu.VMEM((2, CH, width), jnp.float32),
                       pltpu.VMEM((tail,), jnp.int32),
                       pltpu.VMEM((tail, width), jnp.float32),
                       pltpu.VMEM_SHARED((NP, width), jnp.float32),
                       pltpu.SemaphoreType.DMA,
                       pltpu.SemaphoreType.DMA],
    )
    def k(m_hbm, row_hbm, z_hbm, out_hbm, idxs, bufs, tidx, tbuf, acc_sh,
          sr0, sr1):
        c_ax = lax.axis_index("c")
        s_ax = lax.axis_index("s")
        pltpu.sync_copy(z_hbm, acc_sh.at[pl.ds(s_ax * RPT, RPT)])
        plsc.subcore_barrier()

        wbase = (c_ax * NS + s_ax) * epw
        sr = (sr0, sr1)

        def issue_rd(c, p):
            base = pl.multiple_of(wbase + c * CH, 8)
            pltpu.async_copy(row_hbm.at[pl.ds(base, CH)], idxs.at[p], sr[p])
            pltpu.async_copy(m_hbm.at[pl.ds(base, CH)], bufs.at[p], sr[p])

        def wait_rd(p):
            pltpu.make_async_copy(row_hbm.at[pl.ds(0, CH)], idxs.at[p],
                                  sr[p]).wait()
            pltpu.make_async_copy(m_hbm.at[pl.ds(0, CH)], bufs.at[p],
                                  sr[p]).wait()

        issue_rd(0, 0)
        issue_rd(1, 1)

        def body(kk, carry):
            for p in (0, 1):
                c = 2 * kk + p
                wait_rd(p)
                pltpu.sync_copy(bufs.at[p], acc_sh.at[idxs.at[p]], add=True)

                @pl.when(c + 2 < nfull)
                def _():
                    issue_rd(c + 2, p)
            return carry

        lax.fori_loop(0, pairs, body, 0)
        if rem:
            wait_rd(0)
            pltpu.sync_copy(bufs.at[0], acc_sh.at[idxs.at[0]], add=True)

        tb = pl.multiple_of(wbase + nfull * CH, 8)
        pltpu.sync_copy(row_hbm.at[pl.ds(tb, tail)], tidx)
        pltpu.sync_copy(m_hbm.at[pl.ds(tb, tail)], tbuf)
        pltpu.sync_copy(tbuf, acc_sh.at[tidx], add=True)

        plsc.subcore_barrier()
        obase = pl.multiple_of(c_ax * NP + s_ax * RPT, 8)
        pltpu.sync_copy(acc_sh.at[pl.ds(s_ax * RPT, RPT)],
                        out_hbm.at[pl.ds(obase, RPT)])

    return k(vals, row, zeros_tile)


def _tc_edge(xi, xj, pi, pj, edge_attr, eoff,
             w1a, w1b, w1r, w1e, b1, w2, b2, wc1, bc1, wc2r, bc2):
    ecount = xi.shape[0]
    grid = ecount // EB
    boff = eoff // EB

    def body(xi_ref, xj_ref, pi_ref, pj_ref, ea_ref,
             w1a_ref, w1b_ref, w1r_ref, w1e_ref, b1_ref, w2_ref, b2_ref,
             wc1_ref, bc1_ref, wc2_ref, bc2_ref, m_ref, t_ref):
        S = 8
        SB = EB // S
        for s in range(S):
            sl = pl.ds(s * SB, SB)
            pdiff = pi_ref[sl, :] - pj_ref[sl, :]
            radial = jnp.sum(pdiff * pdiff, axis=1, keepdims=True)

            m = (jnp.dot(xi_ref[sl, :], w1a_ref[...],
                         preferred_element_type=jnp.float32)
                 + jnp.dot(xj_ref[sl, :], w1b_ref[...],
                           preferred_element_type=jnp.float32)
                 + jnp.dot(ea_ref[sl, :], w1e_ref[...],
                           preferred_element_type=jnp.float32)
                 + radial * w1r_ref[...]
                 + b1_ref[...])
            m = m * jax.nn.sigmoid(m)
            m = (jnp.dot(m, w2_ref[...], preferred_element_type=jnp.float32)
                 + b2_ref[...])
            m = m * jax.nn.sigmoid(m)
            m_ref[sl, :] = m

            cc = (jnp.dot(m, wc1_ref[...], preferred_element_type=jnp.float32)
                  + bc1_ref[...])
            cc = cc * jax.nn.sigmoid(cc)
            cc = jnp.sum(cc * wc2_ref[...], axis=1, keepdims=True) + bc2_ref[...]

            lane = lax.broadcasted_iota(jnp.int32, (SB, PW), 1)
            t_ref[sl, :] = jnp.where(lane == 3,
                                     jnp.float32(1.0), pdiff * cc)

    full = lambda shape: pl.BlockSpec(shape, lambda i: (0,) * len(shape))
    return pl.pallas_call(
        body,
        grid=(grid,),
        in_specs=[
            pl.BlockSpec((EB, D), lambda i: (i, 0)),
            pl.BlockSpec((EB, D), lambda i: (i, 0)),
            pl.BlockSpec((EB, PW), lambda i: (i, 0)),
            pl.BlockSpec((EB, PW), lambda i: (i, 0)),
            pl.BlockSpec((EB, DE), lambda i: (i + boff, 0)),
            full((D, H)), full((D, H)), full((1, H)), full((DE, H)),
            full((1, H)), full((H, H)), full((1, H)),
            full((H, H)), full((1, H)), full((1, H)), full((1, 1)),
        ],
        out_specs=[pl.BlockSpec((EB, D), lambda i: (i, 0)),
                   pl.BlockSpec((EB, PW), lambda i: (i, 0))],
        out_shape=[jax.ShapeDtypeStruct((ecount, D), jnp.float32),
                   jax.ShapeDtypeStruct((ecount, PW), jnp.float32)],
    )(xi, xj, pi, pj, edge_attr,
      w1a, w1b, w1r, w1e, b1, w2, b2, wc1, bc1, wc2r, bc2)


def _tc_node(mp0, tp0, x, pos, wna, wnb, bn1, wn2, bn2):
    grid = N // NB

    def body(mp0_ref, tp0_ref, x_ref, pos_ref,
             wna_ref, wnb_ref, bn1_ref,
             wn2_ref, bn2_ref, h_ref, np_ref):
        agg = mp0_ref[0] + mp0_ref[1]
        tc16 = tp0_ref[0] + tp0_ref[1]
        sum_trans = tc16[:, :3]
        counts = tc16[:, 3:4]
        xv = x_ref[...]

        h = (jnp.dot(xv, wna_ref[...], preferred_element_type=jnp.float32)
             + jnp.dot(agg, wnb_ref[...], preferred_element_type=jnp.float32)
             + bn1_ref[...])
        h = h * jax.nn.sigmoid(h)
        h = jnp.dot(h, wn2_ref[...], preferred_element_type=jnp.float32) + bn2_ref[...]
        h_ref[...] = xv + h
        np_ref[...] = pos_ref[...] + sum_trans / jnp.maximum(counts, 1.0)

    full = lambda shape: pl.BlockSpec(shape, lambda i: (0,) * len(shape))
    return pl.pallas_call(
        body,
        grid=(grid,),
        in_specs=[
            pl.BlockSpec((NC, NB, D), lambda i: (0, i, 0)),
            pl.BlockSpec((NC, NB, PW), lambda i: (0, i, 0)),
            pl.BlockSpec((NB, D), lambda i: (i, 0)),
            pl.BlockSpec((NB, 3), lambda i: (i, 0)),
            full((D, H)), full((D, H)), full((1, H)),
            full((H, D)), full((1, D)),
        ],
        out_specs=[pl.BlockSpec((NB, D), lambda i: (i, 0)),
                   pl.BlockSpec((NB, 3), lambda i: (i, 0))],
        out_shape=[jax.ShapeDtypeStruct((N, D), jnp.float32),
                   jax.ShapeDtypeStruct((N, 3), jnp.float32)],
    )(mp0, tp0, x, pos, wna, wnb, bn1, wn2, bn2)


def kernel(x, pos, edge_index, edge_attr, W_e1, b_e1, W_e2, b_e2,
           W_c1, b_c1, W_c2, b_c2, W_n1, b_n1, W_n2, b_n2):
    row = edge_index[0]
    col = edge_index[1]

    pos16 = jnp.concatenate([pos, jnp.zeros((N, PW - 3), jnp.float32)], axis=1)

    ew = (W_e1[:D], W_e1[D:2 * D], W_e1[2 * D:2 * D + 1], W_e1[2 * D + 1:],
          b_e1.reshape(1, H), W_e2, b_e2.reshape(1, H),
          W_c1, b_c1.reshape(1, H), W_c2.reshape(1, H), b_c2.reshape(1, 1))
    zm = jnp.zeros((RPT, D), jnp.float32)
    zt = jnp.zeros((RPT, PW), jnp.float32)

    pi, pj = _sc_gather(pos16, row, col, PW, False, E)
    xi, xj = _sc_gather(x, row, col, D, True, E)
    m, tc16 = _tc_edge(xi, xj, pi, pj, edge_attr, 0, *ew)
    mpart = _sc_scatter(m, row, zm, D, True, E).reshape(NC, NP, D)
    tpart = _sc_scatter(tc16, row, zt, PW, False, E).reshape(NC, NP, PW)

    h_out, new_pos = _tc_node(
        mpart, tpart, x, pos,
        W_n1[:D], W_n1[D:], b_n1.reshape(1, H), W_n2, b_n2.reshape(1, D))
    return (h_out, new_pos)
